# baseline (speedup 1.0000x reference)
import functools

import jax
import jax.numpy as jnp
from jax import lax
from jax.experimental import pallas as pl
from jax.experimental.pallas import tpu as pltpu
from jax.experimental.pallas import tpu_sc as plsc

NC = 2
NS = 16
LANES = 16
CHUNK = 80
NBUF = 2


def _matmul_body(x_ref, w_ref, o_ref):
    o_ref[...] = jnp.dot(x_ref[...], w_ref[0],
                         preferred_element_type=jnp.float32)


def _combine_body(a_ref, b_ref, bias_ref, o_ref):
    o_ref[...] = jnp.concatenate([a_ref[0], b_ref[0]], axis=-1) + bias_ref[...]


def _make_spmm(n_nodes, half, per_tile, n_full):
    rows_per_tile = (n_nodes // NS) // 8 * 8
    leftover = n_nodes - rows_per_tile * NS
    assert leftover % 8 == 0
    zrows = rows_per_tile
    for cand in (64, 56, 48, 40, 32, 24, 16, 8):
        if rows_per_tile % cand == 0:
            zrows = cand
            break
    nz = rows_per_tile // zrows
    assert leftover <= zrows
    nv = half // LANES
    assert n_full % NBUF == 0 and CHUNK % LANES == 0

    mesh = plsc.VectorSubcoreMesh(core_axis_name="c", subcore_axis_name="s")

    scratch = [
        pltpu.VMEM((per_tile,), jnp.int32),
        pltpu.VMEM((n_full, CHUNK), jnp.int32),
        pltpu.VMEM((per_tile,), jnp.float32),
        pltpu.VMEM((CHUNK, half), jnp.float32),
        pltpu.VMEM((CHUNK, half), jnp.float32),
        pltpu.VMEM((CHUNK, half), jnp.float32),
        pltpu.VMEM((CHUNK, half), jnp.float32),
        pltpu.VMEM((zrows, half), jnp.float32),
        pltpu.VMEM_SHARED((n_nodes, half), jnp.float32),
        pltpu.SemaphoreType.DMA,
        pltpu.SemaphoreType.DMA,
        pltpu.SemaphoreType.DMA,
        pltpu.SemaphoreType.DMA,
        pltpu.SemaphoreType.DMA,
    ]

    def body(h_hbm, col_hbm, row2d_hbm, w_hbm, out_hbm,
             colv, row2d, w_all, gb0, gb1, sb0, sb1, zblk, agg,
             sstage, sg0, sg1, ss0, ss1):
        gb = (gb0, gb1)
        sb = (sb0, sb1)
        sgat = (sg0, sg1)
        ssc = (ss0, ss1)

        c = lax.axis_index("c")
        s = lax.axis_index("s")
        eb = s * per_tile

        pltpu.async_copy(col_hbm.at[pl.ds(eb, per_tile)], colv, sstage)
        pltpu.async_copy(row2d_hbm.at[pl.ds(s * n_full, n_full)], row2d,
                         sstage)
        pltpu.async_copy(w_hbm.at[pl.ds(eb, per_tile)], w_all, sstage)

        @pl.loop(0, zrows)
        def _(i):
            for j in range(nv):
                zblk[i, pl.ds(j * LANES, LANES)] = jnp.zeros(
                    (LANES,), jnp.float32)

        base_row = s * rows_per_tile
        for q in range(nz):
            pltpu.sync_copy(zblk, agg.at[pl.ds(base_row + q * zrows, zrows)])
        if leftover:
            @pl.when(s == NS - 1)
            def _():
                pltpu.sync_copy(
                    zblk.at[pl.ds(0, leftover)],
                    agg.at[pl.ds(rows_per_tile * NS, leftover)])

        pltpu.make_async_copy(col_hbm.at[pl.ds(eb, per_tile)], colv,
                              sstage).wait()
        pltpu.make_async_copy(row2d_hbm.at[pl.ds(s * n_full, n_full)],
                              row2d, sstage).wait()
        pltpu.make_async_copy(w_hbm.at[pl.ds(eb, per_tile)], w_all,
                              sstage).wait()

        cbase = c * n_nodes

        @pl.loop(0, per_tile // LANES)
        def _(i):
            sl = pl.ds(i * LANES, LANES)
            colv[sl] = colv[sl] + cbase

        plsc.subcore_barrier()

        for b in range(NBUF):
            pltpu.async_copy(
                h_hbm.at[colv.at[pl.ds(b * CHUNK, CHUNK)]], gb[b], sgat[b])

        @pl.loop(0, n_full, step=NBUF)
        def _(g2):
            for b in range(NBUF):
                k = g2 + b

                @pl.when(k >= NBUF)
                def _():
                    pltpu.make_async_copy(
                        sb[b], agg.at[row2d.at[k]], ssc[b]).wait()
                pltpu.make_async_copy(
                    h_hbm.at[colv.at[pl.ds(k * CHUNK, CHUNK)]], gb[b],
                    sgat[b]).wait()

                @pl.loop(0, CHUNK // LANES)
                def _(q):
                    wv16 = w_all[pl.ds(k * CHUNK + q * LANES, LANES)]
                    for l in range(LANES):
                        wb = jnp.full((LANES,), wv16[l], dtype=jnp.float32)
                        e = q * LANES + l
                        for j in range(nv):
                            sl = pl.ds(j * LANES, LANES)
                            sb[b][e, sl] = gb[b][e, sl] * wb

                pltpu.async_copy(sb[b], agg.at[row2d.at[k]], ssc[b],
                                 add=True)

                @pl.when(k + NBUF < n_full)
                def _():
                    pltpu.async_copy(
                        h_hbm.at[colv.at[pl.ds((k + NBUF) * CHUNK, CHUNK)]],
                        gb[b], sgat[b])

        for b in range(NBUF):
            pltpu.make_async_copy(sb[b], agg.at[row2d.at[0]], ssc[b]).wait()

        plsc.subcore_barrier()
        pltpu.sync_copy(agg.at[pl.ds(base_row, rows_per_tile)],
                        out_hbm.at[c, pl.ds(base_row, rows_per_tile)])
        if leftover:
            @pl.when(s == NS - 1)
            def _():
                pltpu.sync_copy(
                    agg.at[pl.ds(rows_per_tile * NS, leftover)],
                    out_hbm.at[c, pl.ds(rows_per_tile * NS, leftover)])

    return pl.kernel(
        body,
        out_type=jax.ShapeDtypeStruct((NC, n_nodes, half), jnp.float32),
        mesh=mesh,
        scratch_types=scratch,
        compiler_params=pltpu.CompilerParams(use_tc_tiling_on_sc=False),
    )


@jax.jit
def kernel(x, edge_index, edge_weight, weight, bias):
    n, d_in = x.shape
    d = weight.shape[1]
    n_edges = edge_weight.shape[0]
    assert d % (2 * LANES) == 0
    half = d // NC

    blk = 1000 if n % 1000 == 0 else n
    nb = n // blk
    w3 = weight.reshape(d_in, NC, half).transpose(1, 0, 2)
    h = pl.pallas_call(
        _matmul_body,
        grid=(NC, nb),
        in_specs=[
            pl.BlockSpec((blk, d_in), lambda c, i: (i, 0)),
            pl.BlockSpec((1, d_in, half), lambda c, i: (c, 0, 0)),
        ],
        out_specs=pl.BlockSpec((blk, half), lambda c, i: (c * nb + i, 0)),
        out_shape=jax.ShapeDtypeStruct((NC * n, half), jnp.float32),
    )(x, w3)

    q = CHUNK * 8
    per_tile = -(-n_edges // (NS * q)) * q
    n_full = per_tile // CHUNK
    assert n_full % 8 == 0 and n_full % NBUF == 0
    total = per_tile * NS

    ei = edge_index.astype(jnp.int32)
    ew = edge_weight.astype(jnp.float32)
    pad = total - n_edges
    if pad:
        row = jnp.concatenate([ei[0], jnp.zeros((pad,), jnp.int32)])
        col = jnp.concatenate([ei[1], jnp.zeros((pad,), jnp.int32)])
        w = jnp.concatenate([ew, jnp.zeros((pad,), jnp.float32)])
    else:
        row, col, w = ei[0], ei[1], ew
    row2d = row.reshape(NS * n_full, CHUNK)

    partials = _make_spmm(n, half, per_tile, n_full)(h, col, row2d, w)

    out = pl.pallas_call(
        _combine_body,
        grid=(nb,),
        in_specs=[
            pl.BlockSpec((1, blk, half), lambda i: (0, i, 0)),
            pl.BlockSpec((1, blk, half), lambda i: (1, i, 0)),
            pl.BlockSpec((d,), lambda i: (0,)),
        ],
        out_specs=pl.BlockSpec((blk, d), lambda i: (i, 0)),
        out_shape=jax.ShapeDtypeStruct((n, d), jnp.float32),
    )(partials, partials, bias)
    return out
